# baseline (speedup 1.0000x reference)
import functools

import jax
import jax.numpy as jnp
from jax import lax
from jax.experimental import pallas as pl
from jax.experimental.pallas import tpu as pltpu

_TRUNC = 32
_N = 8192
_BLK = 128
_BISECT_ITERS = 34


def _build_z(x):
    b, n = x.shape
    lo = jnp.zeros((b, 1), jnp.float32)
    hi = jnp.ones((b, 1), jnp.float32)
    kf = jnp.float32(_TRUNC)

    def bis(_, carry):
        lo, hi = carry
        mid = 0.5 * (lo + hi)
        cnt = jnp.sum(jnp.where(x > mid, 1.0, 0.0), axis=1, keepdims=True)
        ge = cnt >= kf
        return jnp.where(ge, mid, lo), jnp.where(ge, hi, mid)

    lo, hi = lax.fori_loop(0, _BISECT_ITERS, bis, (lo, hi))
    v32 = jnp.min(jnp.where(x > lo, x, 2.0), axis=1, keepdims=True)
    gt = x > v32
    cgt = jnp.sum(jnp.where(gt, 1.0, 0.0), axis=1, keepdims=True)
    need = kf - cgt
    tie = x == v32
    pre = jnp.where(tie, 1.0, 0.0)
    d = 1
    while d < n:
        pre = pre + jnp.concatenate(
            [jnp.zeros((b, d), jnp.float32), pre[:, :-d]], axis=1)
        d *= 2
    sel = gt | (tie & (pre <= need))
    logv = jnp.clip(jnp.log(x), -1000.0, None) + 50.0
    minlog = jnp.clip(jnp.log(v32), -1000.0, None) + 50.0
    shift = jax.nn.relu(-minlog)
    z = jnp.where(sel, logv + shift, 0.0)
    norm = jnp.sqrt(jnp.sum(z * z, axis=1, keepdims=True))
    return z / jnp.clip(norm, 1e-12, None)


_NSTREAM = 4


def _fused_kernel(x_ref, *refs):
    w1_refs = refs[0:_NSTREAM]
    b1_refs = refs[_NSTREAM:2 * _NSTREAM]
    w2_refs = refs[2 * _NSTREAM:3 * _NSTREAM]
    b2_ref = refs[3 * _NSTREAM]
    out_ref = refs[3 * _NSTREAM + 1]
    z_ref, acc_ref = refs[3 * _NSTREAM + 2:]
    j = pl.program_id(0)

    @pl.when(j == 0)
    def _():
        z_ref[...] = _build_z(x_ref[...])
        acc_ref[...] = jnp.zeros_like(acc_ref)

    acc = acc_ref[...]
    for w1_ref, b1_ref, w2_ref in zip(w1_refs, b1_refs, w2_refs):
        h = lax.dot_general(
            z_ref[...], w1_ref[...], (((1,), (1,)), ((), ())),
            preferred_element_type=jnp.float32) + b1_ref[...][None, :]
        h = jnp.where(h >= 0, h, 0.2 * h)
        acc = acc + lax.dot_general(
            h, w2_ref[...], (((1,), (1,)), ((), ())),
            preferred_element_type=jnp.float32)
    acc_ref[...] = acc

    @pl.when(j == pl.num_programs(0) - 1)
    def _():
        out_ref[...] = acc_ref[...] + b2_ref[...][None, :]


@functools.partial(jax.jit, static_argnames=("interpret",))
def _impl(x, W1, b1, W2, b2, interpret=False):
    batch, n = x.shape
    out_dim = W2.shape[0]
    steps = n // _BLK // _NSTREAM

    def _w1_spec(s):
        return pl.BlockSpec((_BLK, n), lambda j, s=s: (j + s * steps, 0))

    def _b1_spec(s):
        return pl.BlockSpec((_BLK,), lambda j, s=s: (j + s * steps,))

    def _w2_spec(s):
        return pl.BlockSpec((out_dim, _BLK), lambda j, s=s: (0, j + s * steps))

    return pl.pallas_call(
        _fused_kernel,
        grid=(steps,),
        in_specs=(
            [pl.BlockSpec((batch, n), lambda j: (0, 0))]
            + [_w1_spec(s) for s in range(_NSTREAM)]
            + [_b1_spec(s) for s in range(_NSTREAM)]
            + [_w2_spec(s) for s in range(_NSTREAM)]
            + [pl.BlockSpec((out_dim,), lambda j: (0,))]
        ),
        out_specs=pl.BlockSpec((batch, out_dim), lambda j: (0, 0)),
        out_shape=jax.ShapeDtypeStruct((batch, out_dim), jnp.float32),
        scratch_shapes=[
            pltpu.VMEM((batch, n), jnp.float32),
            pltpu.VMEM((batch, out_dim), jnp.float32),
        ],
        interpret=interpret,
    )(x, *([W1] * _NSTREAM), *([b1] * _NSTREAM), *([W2] * _NSTREAM), b2)


def kernel(x, W1, b1, W2, b2):
    return _impl(x, W1, b1, W2, b2)


# speedup vs baseline: 1.0495x; 1.0495x over previous
import functools

import jax
import jax.numpy as jnp
from jax import lax
from jax.experimental import pallas as pl
from jax.experimental.pallas import tpu as pltpu

_TRUNC = 32
_N = 8192
_BLK = 256
_BISECT_ITERS = 34


def _build_z(x):
    b, n = x.shape
    lo = jnp.zeros((b, 1), jnp.float32)
    hi = jnp.ones((b, 1), jnp.float32)
    kf = jnp.float32(_TRUNC)

    def bis(_, carry):
        lo, hi = carry
        mid = 0.5 * (lo + hi)
        cnt = jnp.sum(jnp.where(x > mid, 1.0, 0.0), axis=1, keepdims=True)
        ge = cnt >= kf
        return jnp.where(ge, mid, lo), jnp.where(ge, hi, mid)

    lo, hi = lax.fori_loop(0, _BISECT_ITERS, bis, (lo, hi))
    v32 = jnp.min(jnp.where(x > lo, x, 2.0), axis=1, keepdims=True)
    gt = x > v32
    cgt = jnp.sum(jnp.where(gt, 1.0, 0.0), axis=1, keepdims=True)
    need = kf - cgt
    tie = x == v32
    pre = jnp.where(tie, 1.0, 0.0)
    d = 1
    while d < n:
        pre = pre + jnp.concatenate(
            [jnp.zeros((b, d), jnp.float32), pre[:, :-d]], axis=1)
        d *= 2
    sel = gt | (tie & (pre <= need))
    logv = jnp.clip(jnp.log(x), -1000.0, None) + 50.0
    minlog = jnp.clip(jnp.log(v32), -1000.0, None) + 50.0
    shift = jax.nn.relu(-minlog)
    z = jnp.where(sel, logv + shift, 0.0)
    norm = jnp.sqrt(jnp.sum(z * z, axis=1, keepdims=True))
    return z / jnp.clip(norm, 1e-12, None)


def _fused_kernel(x_ref, w1a_ref, w1b_ref, b1a_ref, b1b_ref,
                  w2a_ref, w2b_ref, b2_ref, out_ref, z_ref, acc_ref):
    j = pl.program_id(0)

    @pl.when(j == 0)
    def _():
        z_ref[...] = _build_z(x_ref[...])
        acc_ref[...] = jnp.zeros_like(acc_ref)

    acc = acc_ref[...]
    for w1_ref, b1_ref, w2_ref in ((w1a_ref, b1a_ref, w2a_ref),
                                   (w1b_ref, b1b_ref, w2b_ref)):
        h = lax.dot_general(
            z_ref[...], w1_ref[...], (((1,), (1,)), ((), ())),
            preferred_element_type=jnp.float32) + b1_ref[...][None, :]
        h = jnp.where(h >= 0, h, 0.2 * h)
        acc = acc + lax.dot_general(
            h, w2_ref[...], (((1,), (1,)), ((), ())),
            preferred_element_type=jnp.float32)
    acc_ref[...] = acc

    @pl.when(j == pl.num_programs(0) - 1)
    def _():
        out_ref[...] = acc_ref[...] + b2_ref[...][None, :]


@functools.partial(jax.jit, static_argnames=("interpret",))
def _impl(x, W1, b1, W2, b2, interpret=False):
    batch, n = x.shape
    out_dim = W2.shape[0]
    half_steps = n // _BLK // 2
    return pl.pallas_call(
        _fused_kernel,
        grid=(half_steps,),
        in_specs=[
            pl.BlockSpec((batch, n), lambda j: (0, 0)),
            pl.BlockSpec((_BLK, n), lambda j: (j, 0)),
            pl.BlockSpec((_BLK, n), lambda j: (j + half_steps, 0)),
            pl.BlockSpec((_BLK,), lambda j: (j,)),
            pl.BlockSpec((_BLK,), lambda j: (j + half_steps,)),
            pl.BlockSpec((out_dim, _BLK), lambda j: (0, j)),
            pl.BlockSpec((out_dim, _BLK), lambda j: (0, j + half_steps)),
            pl.BlockSpec((out_dim,), lambda j: (0,)),
        ],
        out_specs=pl.BlockSpec((batch, out_dim), lambda j: (0, 0)),
        out_shape=jax.ShapeDtypeStruct((batch, out_dim), jnp.float32),
        scratch_shapes=[
            pltpu.VMEM((batch, n), jnp.float32),
            pltpu.VMEM((batch, out_dim), jnp.float32),
        ],
        interpret=interpret,
    )(x, W1, W1, b1, b1, W2, W2, b2)


def kernel(x, W1, b1, W2, b2):
    return _impl(x, W1, b1, W2, b2)


# speedup vs baseline: 1.0523x; 1.0026x over previous
import jax
import jax.numpy as jnp
from jax import lax
from jax.experimental import pallas as pl
from jax.experimental.pallas import tpu as pltpu

_TRUNC = 32
_N = 8192
_BLK = 256
_BISECT_ITERS = 34


def _build_z(x):
    b, n = x.shape
    lo = jnp.zeros((b, 1), jnp.float32)
    hi = jnp.ones((b, 1), jnp.float32)
    kf = jnp.float32(_TRUNC)

    def bis(_, carry):
        lo, hi = carry
        mid = 0.5 * (lo + hi)
        cnt = jnp.sum(jnp.where(x > mid, 1.0, 0.0), axis=1, keepdims=True)
        ge = cnt >= kf
        return jnp.where(ge, mid, lo), jnp.where(ge, hi, mid)

    lo, hi = lax.fori_loop(0, _BISECT_ITERS, bis, (lo, hi))
    v32 = jnp.min(jnp.where(x > lo, x, 2.0), axis=1, keepdims=True)
    gt = x > v32
    cgt = jnp.sum(jnp.where(gt, 1.0, 0.0), axis=1, keepdims=True)
    need = kf - cgt
    tie = x == v32
    pre = jnp.where(tie, 1.0, 0.0)
    d = 1
    while d < n:
        pre = pre + jnp.concatenate(
            [jnp.zeros((b, d), jnp.float32), pre[:, :-d]], axis=1)
        d *= 2
    sel = gt | (tie & (pre <= need))
    logv = jnp.clip(jnp.log(x), -1000.0, None) + 50.0
    minlog = jnp.clip(jnp.log(v32), -1000.0, None) + 50.0
    shift = jax.nn.relu(-minlog)
    z = jnp.where(sel, logv + shift, 0.0)
    norm = jnp.sqrt(jnp.sum(z * z, axis=1, keepdims=True))
    return z / jnp.clip(norm, 1e-12, None)


def _fused_kernel(x_ref, w1a_ref, w1b_ref, b1a_ref, b1b_ref,
                  w2a_ref, w2b_ref, b2_ref, out_ref, z_ref, acc_ref):
    j = pl.program_id(0)

    @pl.when(j == 0)
    def _():
        z_ref[...] = _build_z(x_ref[...])
        acc_ref[...] = jnp.zeros_like(acc_ref)

    acc = acc_ref[...]
    for w1_ref, b1_ref, w2_ref in ((w1a_ref, b1a_ref, w2a_ref),
                                   (w1b_ref, b1b_ref, w2b_ref)):
        h = lax.dot_general(
            z_ref[...], w1_ref[...], (((1,), (1,)), ((), ())),
            preferred_element_type=jnp.float32) + b1_ref[...][None, :]
        h = jnp.where(h >= 0, h, 0.2 * h)
        acc = acc + lax.dot_general(
            h, w2_ref[...], (((1,), (1,)), ((), ())),
            preferred_element_type=jnp.float32)
    acc_ref[...] = acc

    @pl.when(j == pl.num_programs(0) - 1)
    def _():
        out_ref[...] = acc_ref[...] + b2_ref[...][None, :]


@jax.jit
def _impl(x, W1, b1, W2, b2):
    batch, n = x.shape
    out_dim = W2.shape[0]
    half_steps = n // _BLK // 2
    return pl.pallas_call(
        _fused_kernel,
        grid=(half_steps,),
        in_specs=[
            pl.BlockSpec((batch, n), lambda j: (0, 0)),
            pl.BlockSpec((_BLK, n), lambda j: (j, 0)),
            pl.BlockSpec((_BLK, n), lambda j: (j + half_steps, 0)),
            pl.BlockSpec((_BLK,), lambda j: (j,)),
            pl.BlockSpec((_BLK,), lambda j: (j + half_steps,)),
            pl.BlockSpec((out_dim, _BLK), lambda j: (0, j)),
            pl.BlockSpec((out_dim, _BLK), lambda j: (0, j + half_steps)),
            pl.BlockSpec((out_dim,), lambda j: (0,)),
        ],
        out_specs=pl.BlockSpec((batch, out_dim), lambda j: (0, 0)),
        out_shape=jax.ShapeDtypeStruct((batch, out_dim), jnp.float32),
        scratch_shapes=[
            pltpu.VMEM((batch, n), jnp.float32),
            pltpu.VMEM((batch, out_dim), jnp.float32),
        ],
    )(x, W1, W1, b1, b1, W2, W2, b2)


def kernel(x, W1, b1, W2, b2):
    return _impl(x, W1, b1, W2, b2)
